# baseline (speedup 1.0000x reference)
import jax
import jax.numpy as jnp
from jax import lax
from jax.experimental import pallas as pl
from jax.experimental.pallas import tpu as pltpu
from jax.experimental.pallas import tpu_sc as plsc

_B = 128
_COLS = 100000
_C = 50
_W = 2000
_FLAT = _B * _C
_NUM = 38
_NW = 32
_BLEND_ROWS = 2048
_TASKS_PW = _BLEND_ROWS // _NW
_GROUP = 8
_NGROUP = _TASKS_PW // _GROUP
_COPY_ROWS = _FLAT - _BLEND_ROWS
_COPY_PW = _COPY_ROWS // _NW


def _body(x2, srctab, out, idx_v, a_v, b_v, sem_cp, sem_a, sem_b, sem_w):
    cid = lax.axis_index("c")
    sid = lax.axis_index("s")
    w = sid * 2 + cid

    cp_base = _BLEND_ROWS + w * _COPY_PW
    cp = pltpu.make_async_copy(
        x2.at[pl.ds(cp_base, _COPY_PW)], out.at[pl.ds(cp_base, _COPY_PW)], sem_cp
    )
    cp.start()

    pltpu.sync_copy(srctab.at[w], idx_v)

    base = w * _TASKS_PW
    for g in range(_NGROUP):
        row0 = base + g * _GROUP
        ca = pltpu.make_async_copy(x2.at[pl.ds(row0, _GROUP)], a_v, sem_a)
        ca.start()
        cb = pltpu.make_async_copy(x2.at[idx_v.at[g]], b_v, sem_b)
        cb.start()
        ca.wait()
        cb.wait()

        def _blend(k, carry):
            off = pl.multiple_of(k * 16, 16)
            for rr in range(_GROUP):
                a = a_v[rr, pl.ds(off, 16)]
                b = b_v[rr, pl.ds(off, 16)]
                a_v[rr, pl.ds(off, 16)] = a + (b - a) * 0.5
            return carry

        lax.fori_loop(0, _W // 16, _blend, 0)

        cw = pltpu.make_async_copy(a_v, out.at[pl.ds(row0, _GROUP)], sem_w)
        cw.start()
        cw.wait()

    cp.wait()


_blend_copy = pl.kernel(
    _body,
    out_type=jax.ShapeDtypeStruct((_FLAT, _W), jnp.float32),
    mesh=plsc.VectorSubcoreMesh(core_axis_name="c", subcore_axis_name="s"),
    scratch_types=[
        pltpu.VMEM((_NGROUP, _GROUP), jnp.int32),
        pltpu.VMEM((_GROUP, _W), jnp.float32),
        pltpu.VMEM((_GROUP, _W), jnp.float32),
        pltpu.SemaphoreType.DMA,
        pltpu.SemaphoreType.DMA,
        pltpu.SemaphoreType.DMA,
        pltpu.SemaphoreType.DMA,
    ],
    compiler_params=pltpu.CompilerParams(use_tc_tiling_on_sc=False),
)


def kernel(x):
    x2 = x.reshape(_FLAT, _W)
    i = jnp.arange(_B, dtype=jnp.int32)
    r = jax.random.randint(jax.random.key(1234), (_B,), 0, _B, dtype=jnp.int32)
    s = jnp.where((i >= 1) & (i < _NUM), r, i)
    j = jnp.arange(_BLEND_ROWS, dtype=jnp.int32)
    src = s[j // _C] * _C + j % _C
    srctab = src.reshape(_NW, _NGROUP, _GROUP)
    out2 = _blend_copy(x2, srctab)
    return out2.reshape(_B, _COLS)


# speedup vs baseline: 4.4739x; 4.4739x over previous
import jax
import jax.numpy as jnp
from jax import lax
from jax.experimental import pallas as pl
from jax.experimental.pallas import tpu as pltpu
from jax.experimental.pallas import tpu_sc as plsc

_B = 128
_COLS = 100000
_C = 50
_W = 2000
_FLAT = _B * _C
_NUM = 38
_NW = 32

_NB = 12
_BLEND_ROWS = 1920
_ROWS_PB = _BLEND_ROWS // _NB
_G = 8
_NGRP = _ROWS_PB // _G
_NPAIR = _NGRP // 2

_NCP = _NW - _NB
_COPY_ROWS = _FLAT - _BLEND_ROWS
_ROWS_PC = _COPY_ROWS // _NCP
_NCHUNK = _ROWS_PC // _G
_NQUAD = _NCHUNK // 4


def _body(x2, srctab, out, idx_v, a0, b0, a1, b1,
          sla0, slb0, sla1, slb1, sst0, sst1, sst2, sst3):
    cid = lax.axis_index("c")
    sid = lax.axis_index("s")
    w = sid * 2 + cid

    @pl.when(w < _NB)
    def _blend_path():
        pltpu.sync_copy(srctab.at[w], idx_v)
        base = w * _ROWS_PB

        def _load(g, av, bv, sa, sb):
            row0 = base + g * _G
            pltpu.make_async_copy(x2.at[pl.ds(row0, _G)], av, sa).start()
            pltpu.make_async_copy(x2.at[idx_v.at[g]], bv, sb).start()

        def _wait_load(av, bv, sa, sb):
            pltpu.make_async_copy(x2.at[pl.ds(0, _G)], av, sa).wait()
            pltpu.make_async_copy(x2.at[pl.ds(0, _G)], bv, sb).wait()

        def _store(g, av, ss):
            row0 = base + g * _G
            pltpu.make_async_copy(av, out.at[pl.ds(row0, _G)], ss).start()

        def _wait_store(av, ss):
            pltpu.make_async_copy(av, out.at[pl.ds(0, _G)], ss).wait()

        def _compute(av, bv):
            def _k(k, carry):
                off = pl.multiple_of(k * 16, 16)
                for rr in range(_G):
                    a = av[rr, pl.ds(off, 16)]
                    b = bv[rr, pl.ds(off, 16)]
                    av[rr, pl.ds(off, 16)] = a + (b - a) * 0.5
                return carry

            lax.fori_loop(0, _W // 16, _k, 0)

        _load(0, a0, b0, sla0, slb0)

        def _pair(i, carry):
            g0 = 2 * i
            @pl.when(i > 0)
            def _():
                _wait_store(a1, sst1)

            _load(g0 + 1, a1, b1, sla1, slb1)
            _wait_load(a0, b0, sla0, slb0)
            _compute(a0, b0)
            _store(g0, a0, sst0)

            @pl.when(i < _NPAIR - 1)
            def _():
                _wait_store(a0, sst0)
                _load(g0 + 2, a0, b0, sla0, slb0)

            _wait_load(a1, b1, sla1, slb1)
            _compute(a1, b1)
            _store(g0 + 1, a1, sst1)
            return carry

        lax.fori_loop(0, _NPAIR, _pair, 0)
        _wait_store(a0, sst0)
        _wait_store(a1, sst1)

    @pl.when(w >= _NB)
    def _copy_path():
        cbase = _BLEND_ROWS + (w - _NB) * _ROWS_PC
        bufs = (a0, a1, b0, b1)
        lsems = (sla0, sla1, slb0, slb1)
        ssems = (sst0, sst1, sst2, sst3)

        def _quad(i, carry):
            for k in range(4):
                @pl.when(i > 0)
                def _(k=k):
                    pltpu.make_async_copy(
                        bufs[k], out.at[pl.ds(0, _G)], ssems[k]
                    ).wait()

                row0 = cbase + (4 * i + k) * _G
                pltpu.make_async_copy(
                    x2.at[pl.ds(row0, _G)], bufs[k], lsems[k]
                ).start()
            for k in range(4):
                row0 = cbase + (4 * i + k) * _G
                pltpu.make_async_copy(
                    x2.at[pl.ds(0, _G)], bufs[k], lsems[k]
                ).wait()
                pltpu.make_async_copy(
                    bufs[k], out.at[pl.ds(row0, _G)], ssems[k]
                ).start()
            return carry

        lax.fori_loop(0, _NQUAD, _quad, 0)
        for k in range(4):
            pltpu.make_async_copy(bufs[k], out.at[pl.ds(0, _G)], ssems[k]).wait()


_blend_copy = pl.kernel(
    _body,
    out_type=jax.ShapeDtypeStruct((_FLAT, _W), jnp.float32),
    mesh=plsc.VectorSubcoreMesh(core_axis_name="c", subcore_axis_name="s"),
    scratch_types=[
        pltpu.VMEM((_NGRP, _G), jnp.int32),
        pltpu.VMEM((_G, _W), jnp.float32),
        pltpu.VMEM((_G, _W), jnp.float32),
        pltpu.VMEM((_G, _W), jnp.float32),
        pltpu.VMEM((_G, _W), jnp.float32),
        pltpu.SemaphoreType.DMA,
        pltpu.SemaphoreType.DMA,
        pltpu.SemaphoreType.DMA,
        pltpu.SemaphoreType.DMA,
        pltpu.SemaphoreType.DMA,
        pltpu.SemaphoreType.DMA,
        pltpu.SemaphoreType.DMA,
        pltpu.SemaphoreType.DMA,
    ],
    compiler_params=pltpu.CompilerParams(use_tc_tiling_on_sc=False),
)


def kernel(x):
    x2 = x.reshape(_FLAT, _W)
    i = jnp.arange(_B, dtype=jnp.int32)
    r = jax.random.randint(jax.random.key(1234), (_B,), 0, _B, dtype=jnp.int32)
    s = jnp.where((i >= 1) & (i < _NUM), r, i)
    j = jnp.arange(_BLEND_ROWS, dtype=jnp.int32)
    src = jnp.where(
        j < _NUM * _C, s[j // _C] * _C + j % _C, j
    )
    srctab = src.reshape(_NB, _NGRP, _G)
    out2 = _blend_copy(x2, srctab)
    return out2.reshape(_B, _COLS)


# speedup vs baseline: 7.7741x; 1.7377x over previous
import jax
import jax.numpy as jnp
import numpy as np
from jax import lax
from jax.experimental import pallas as pl
from jax.experimental.pallas import tpu as pltpu
from jax.experimental.pallas import tpu_sc as plsc

_B = 128
_COLS = 100000
_NUM = 38
_PW = 1408
_NP = 71
_TAIL0 = _NP * _PW
_TAILW = _COLS - _TAIL0
_G = 8
_NRG = _B // _G
_NBG = 5
_NT = _NRG * _NP
_NW = 32
_TPW = 36
_NPAIR = _TPW // 2

_R_STATIC = np.asarray([
    53, 33, 93, 32, 28, 102, 38, 94, 65, 35, 96, 98, 48, 45, 76, 124, 120,
    41, 107, 4, 74, 32, 82, 107, 76, 21, 119, 50, 127, 90, 55, 21, 97, 108,
    17, 24, 42, 114, 40, 120, 5, 109, 84, 15, 2, 22, 60, 28, 77, 124, 105,
    83, 43, 94, 21, 36, 71, 124, 0, 2, 108, 11, 44, 119, 81, 31, 71, 110,
    119, 6, 58, 102, 12, 54, 5, 67, 68, 14, 97, 109, 113, 39, 97, 100, 109,
    28, 126, 13, 53, 41, 120, 14, 33, 49, 100, 70, 30, 27, 17, 13, 46, 78,
    40, 120, 63, 76, 116, 7, 53, 33, 55, 60, 42, 57, 86, 51, 108, 4, 111,
    107, 34, 10, 110, 124, 97, 51, 25, 122,
], dtype=np.int32)
_S_STATIC = np.arange(_B)
_S_STATIC[1:_NUM] = _R_STATIC[1:_NUM]


def _body(x, idxtab, out, idx_v, a0, b0, a1, b1, tin_v, tbl_v,
          sla0, slb0, sla1, slb1, sst0, sst1, stl):
    cid = lax.axis_index("c")
    sid = lax.axis_index("s")
    w = sid * 2 + cid

    pltpu.sync_copy(idxtab, idx_v)

    def rgp(t):
        t_e = jnp.minimum(t, _NT - 1)
        rg = t_e // _NP
        p = t_e - rg * _NP
        return rg, p

    def load(t, av, bv, sa, sb):
        rg, p = rgp(t)
        row0 = rg * _G
        c0 = p * _PW
        pltpu.make_async_copy(
            x.at[pl.ds(row0, _G), pl.ds(c0, _PW)], av, sa
        ).start()

        @pl.when(rg < _NBG)
        def _():
            xs = x.at[:, pl.ds(c0, _PW)]
            pltpu.make_async_copy(xs.at[idx_v.at[rg]], bv, sb).start()

    def wait_load(t, av, bv, sa, sb):
        rg, _ = rgp(t)
        pltpu.make_async_copy(x.at[pl.ds(0, _G), pl.ds(0, _PW)], av, sa).wait()

        @pl.when(rg < _NBG)
        def _():
            pltpu.make_async_copy(
                x.at[pl.ds(0, _G), pl.ds(0, _PW)], bv, sb
            ).wait()

    def compute_store(t, av, bv, ss):
        rg, p = rgp(t)

        @pl.when(rg < _NBG)
        def _():
            def _k(k, carry):
                off = pl.multiple_of(k * 16, 16)
                for rr in range(_G):
                    a = av[rr, pl.ds(off, 16)]
                    b = bv[rr, pl.ds(off, 16)]
                    av[rr, pl.ds(off, 16)] = a + (b - a) * 0.5
                return carry

            lax.fori_loop(0, _PW // 16, _k, 0)

        row0 = rg * _G
        c0 = p * _PW
        pltpu.make_async_copy(
            av, out.at[pl.ds(row0, _G), pl.ds(c0, _PW)], ss
        ).start()

    def wait_store(av, ss):
        pltpu.make_async_copy(
            av, out.at[pl.ds(0, _G), pl.ds(0, _PW)], ss
        ).wait()

    @pl.when(w == _NW - 1)
    def _tail():
        pltpu.make_async_copy(
            x.at[pl.ds(0, _B), pl.ds(_TAIL0, _TAILW)], tin_v, stl
        ).start()
        pltpu.make_async_copy(
            x.at[pl.ds(0, _B), pl.ds(_TAIL0, _TAILW)], tin_v, stl
        ).wait()
        for i in range(_NBG * _G):
            pi = int(_S_STATIC[i])
            for off in (0, 16):
                a = tin_v[i, pl.ds(off, 16)]
                b = tin_v[pi, pl.ds(off, 16)]
                tbl_v[i, pl.ds(off, 16)] = a + (b - a) * 0.5
        pltpu.make_async_copy(
            tbl_v, out.at[pl.ds(0, _NBG * _G), pl.ds(_TAIL0, _TAILW)], stl
        ).start()
        pltpu.make_async_copy(
            tin_v.at[pl.ds(_NBG * _G, _B - _NBG * _G)],
            out.at[pl.ds(_NBG * _G, _B - _NBG * _G), pl.ds(_TAIL0, _TAILW)],
            stl,
        ).start()
        pltpu.make_async_copy(
            tbl_v, out.at[pl.ds(0, _NBG * _G), pl.ds(_TAIL0, _TAILW)], stl
        ).wait()
        pltpu.make_async_copy(
            tin_v.at[pl.ds(_NBG * _G, _B - _NBG * _G)],
            out.at[pl.ds(_NBG * _G, _B - _NBG * _G), pl.ds(_TAIL0, _TAILW)],
            stl,
        ).wait()

    load(w, a0, b0, sla0, slb0)

    def _pair(k2, carry):
        t0 = w + (2 * k2) * _NW
        t1 = t0 + _NW

        @pl.when(k2 > 0)
        def _():
            wait_store(a1, sst1)

        load(t1, a1, b1, sla1, slb1)
        wait_load(t0, a0, b0, sla0, slb0)
        compute_store(t0, a0, b0, sst0)

        @pl.when(k2 < _NPAIR - 1)
        def _():
            wait_store(a0, sst0)
            load(t0 + 2 * _NW, a0, b0, sla0, slb0)

        wait_load(t1, a1, b1, sla1, slb1)
        compute_store(t1, a1, b1, sst1)
        return carry

    lax.fori_loop(0, _NPAIR, _pair, 0)
    wait_store(a0, sst0)
    wait_store(a1, sst1)


_blend_copy = pl.kernel(
    _body,
    out_type=jax.ShapeDtypeStruct((_B, _COLS), jnp.float32),
    mesh=plsc.VectorSubcoreMesh(core_axis_name="c", subcore_axis_name="s"),
    scratch_types=[
        pltpu.VMEM((_G, _G), jnp.int32),
        pltpu.VMEM((_G, _PW), jnp.float32),
        pltpu.VMEM((_G, _PW), jnp.float32),
        pltpu.VMEM((_G, _PW), jnp.float32),
        pltpu.VMEM((_G, _PW), jnp.float32),
        pltpu.VMEM((_B, _TAILW), jnp.float32),
        pltpu.VMEM((_NBG * _G, _TAILW), jnp.float32),
        pltpu.SemaphoreType.DMA,
        pltpu.SemaphoreType.DMA,
        pltpu.SemaphoreType.DMA,
        pltpu.SemaphoreType.DMA,
        pltpu.SemaphoreType.DMA,
        pltpu.SemaphoreType.DMA,
        pltpu.SemaphoreType.DMA,
    ],
    compiler_params=pltpu.CompilerParams(use_tc_tiling_on_sc=True),
)


def kernel(x):
    s = jnp.asarray(_S_STATIC, dtype=jnp.int32)
    idxtab = jnp.zeros((_G, _G), jnp.int32).at[:_NBG].set(
        s[: _NBG * _G].reshape(_NBG, _G)
    )
    return _blend_copy(x, idxtab)


# speedup vs baseline: 14.4491x; 1.8586x over previous
import jax
import jax.numpy as jnp
import numpy as np
from jax import lax
from jax.experimental import pallas as pl
from jax.experimental.pallas import tpu as pltpu
from jax.experimental.pallas import tpu_sc as plsc

_B = 128
_COLS = 100000
_NUM = 38
_NW = 32
_R = 200
_BLK = _R * _B
_NT = (_B * _COLS) // _BLK
_TPW = 16
_NPAIR = _TPW // 2
_NSL = 3

_R_STATIC = np.asarray([
    53, 33, 93, 32, 28, 102, 38, 94, 65, 35, 96, 98, 48, 45, 76, 124, 120,
    41, 107, 4, 74, 32, 82, 107, 76, 21, 119, 50, 127, 90, 55, 21, 97, 108,
    17, 24, 42, 114, 40, 120, 5, 109, 84, 15, 2, 22, 60, 28, 77, 124, 105,
    83, 43, 94, 21, 36, 71, 124, 0, 2, 108, 11, 44, 119, 81, 31, 71, 110,
    119, 6, 58, 102, 12, 54, 5, 67, 68, 14, 97, 109, 113, 39, 97, 100, 109,
    28, 126, 13, 53, 41, 120, 14, 33, 49, 100, 70, 30, 27, 17, 13, 46, 78,
    40, 120, 63, 76, 116, 7, 53, 33, 55, 60, 42, 57, 86, 51, 108, 4, 111,
    107, 34, 10, 110, 124, 97, 51, 25, 122,
], dtype=np.int32)
_S_STATIC = np.arange(_B, dtype=np.int32)
_S_STATIC[1:_NUM] = _R_STATIC[1:_NUM]


def _body(x1, stab, out, sv, f0, f1, sl0, sl1, ss0, ss1):
    cid = lax.axis_index("c")
    sid = lax.axis_index("s")
    w = sid * 2 + cid

    pltpu.sync_copy(stab, sv)

    def teff(t):
        return jnp.where(t < _NT, t, w)

    def load(t, fv, sl):
        q0 = teff(t) * _BLK
        pltpu.make_async_copy(x1.at[pl.ds(q0, _BLK)], fv, sl).start()

    def wait_load(fv, sl):
        pltpu.make_async_copy(x1.at[pl.ds(0, _BLK)], fv, sl).wait()

    def compute(fv):
        def row(rr, carry):
            base = rr * _B
            bs = []
            for k in range(_NSL):
                idx = sv[k, :] + base
                bs.append(plsc.load_gather(fv, [idx]))
            for k in range(_NSL):
                off = base + 16 * k
                a = fv[pl.ds(off, 16)]
                fv[pl.ds(off, 16)] = a + (bs[k] - a) * 0.5
            return carry

        lax.fori_loop(0, _R, row, 0)

    def store(t, fv, ss):
        q0 = teff(t) * _BLK
        pltpu.make_async_copy(fv, out.at[pl.ds(q0, _BLK)], ss).start()

    def wait_store(fv, ss):
        pltpu.make_async_copy(fv, out.at[pl.ds(0, _BLK)], ss).wait()

    load(w, f0, sl0)

    def pair(k2, carry):
        t0 = w + (2 * k2) * _NW
        t1 = t0 + _NW

        @pl.when(k2 > 0)
        def _():
            wait_store(f1, ss1)

        load(t1, f1, sl1)
        wait_load(f0, sl0)
        compute(f0)
        store(t0, f0, ss0)

        @pl.when(k2 < _NPAIR - 1)
        def _():
            wait_store(f0, ss0)
            load(t0 + 2 * _NW, f0, sl0)

        wait_load(f1, sl1)
        compute(f1)
        store(t1, f1, ss1)
        return carry

    lax.fori_loop(0, _NPAIR, pair, 0)
    wait_store(f0, ss0)
    wait_store(f1, ss1)


_blend = pl.kernel(
    _body,
    out_type=jax.ShapeDtypeStruct((_B * _COLS,), jnp.float32),
    mesh=plsc.VectorSubcoreMesh(core_axis_name="c", subcore_axis_name="s"),
    scratch_types=[
        pltpu.VMEM((8, 16), jnp.int32),
        pltpu.VMEM((_BLK,), jnp.float32),
        pltpu.VMEM((_BLK,), jnp.float32),
        pltpu.SemaphoreType.DMA,
        pltpu.SemaphoreType.DMA,
        pltpu.SemaphoreType.DMA,
        pltpu.SemaphoreType.DMA,
    ],
    compiler_params=pltpu.CompilerParams(
        use_tc_tiling_on_sc=True, needs_layout_passes=False
    ),
)


def kernel(x):
    x1 = x.T.reshape(_B * _COLS)
    stab = jnp.zeros((8, 16), jnp.int32).at[:_NSL].set(
        jnp.asarray(_S_STATIC[: _NSL * 16].reshape(_NSL, 16))
    )
    out1 = _blend(x1, stab)
    return out1.reshape(_COLS, _B).T


# speedup vs baseline: 21.4534x; 1.4848x over previous
import jax
import jax.numpy as jnp
import numpy as np
from jax import lax
from jax.experimental import pallas as pl
from jax.experimental.pallas import tpu as pltpu
from jax.experimental.pallas import tpu_sc as plsc

_B = 128
_COLS = 100000
_NUM = 38
_NW = 32
_R = 200
_BLK = _R * _B
_NT = (_B * _COLS) // _BLK
_TPW = 16
_NPAIR = _TPW // 2
_NSL = 3

_R_STATIC = np.asarray([
    53, 33, 93, 32, 28, 102, 38, 94, 65, 35, 96, 98, 48, 45, 76, 124, 120,
    41, 107, 4, 74, 32, 82, 107, 76, 21, 119, 50, 127, 90, 55, 21, 97, 108,
    17, 24, 42, 114, 40, 120, 5, 109, 84, 15, 2, 22, 60, 28, 77, 124, 105,
    83, 43, 94, 21, 36, 71, 124, 0, 2, 108, 11, 44, 119, 81, 31, 71, 110,
    119, 6, 58, 102, 12, 54, 5, 67, 68, 14, 97, 109, 113, 39, 97, 100, 109,
    28, 126, 13, 53, 41, 120, 14, 33, 49, 100, 70, 30, 27, 17, 13, 46, 78,
    40, 120, 63, 76, 116, 7, 53, 33, 55, 60, 42, 57, 86, 51, 108, 4, 111,
    107, 34, 10, 110, 124, 97, 51, 25, 122,
], dtype=np.int32)
_S_STATIC = np.arange(_B, dtype=np.int32)
_S_STATIC[1:_NUM] = _R_STATIC[1:_NUM]


def _body(x1, stab, out, sv, f0, f1, f2, f3,
          sl0, sl1, sl2, sl3, ss0, ss1, ss2, ss3):
    cid = lax.axis_index("c")
    sid = lax.axis_index("s")
    w = sid * 2 + cid

    pltpu.sync_copy(stab, sv)

    fs = (f0, f1, f2, f3)
    sls = (sl0, sl1, sl2, sl3)
    sss = (ss0, ss1, ss2, ss3)

    def teff(t):
        return jnp.where(t < _NT, t, w)

    def load(k, j):
        q0 = teff(w + k * _NW) * _BLK
        pltpu.make_async_copy(x1.at[pl.ds(q0, _BLK)], fs[j], sls[j]).start()

    def wait_load(j):
        pltpu.make_async_copy(x1.at[pl.ds(0, _BLK)], fs[j], sls[j]).wait()

    def compute(j):
        fv = fs[j]
        svs = [sv[k, :] for k in range(_NSL)]

        def row(rr, carry):
            base = rr * _B
            bs = [plsc.load_gather(fv, [svs[k] + base]) for k in range(_NSL)]
            for k in range(_NSL):
                off = base + 16 * k
                a = fv[pl.ds(off, 16)]
                fv[pl.ds(off, 16)] = a + (bs[k] - a) * 0.5
            return carry

        lax.fori_loop(0, _R, row, 0)

    def store(k, j):
        q0 = teff(w + k * _NW) * _BLK
        pltpu.make_async_copy(fs[j], out.at[pl.ds(q0, _BLK)], sss[j]).start()

    def wait_store(j):
        pltpu.make_async_copy(fs[j], out.at[pl.ds(0, _BLK)], sss[j]).wait()

    load(0, 0)
    load(1, 1)
    for k in range(_TPW):
        j = k % 4
        if k + 2 < _TPW:
            if k >= 2:
                wait_store((k + 2) % 4)
            load(k + 2, (k + 2) % 4)
        wait_load(j)
        compute(j)
        store(k, j)
    for j in range(4):
        wait_store(j)


_blend = pl.kernel(
    _body,
    out_type=jax.ShapeDtypeStruct((_B * _COLS,), jnp.float32),
    mesh=plsc.VectorSubcoreMesh(core_axis_name="c", subcore_axis_name="s"),
    scratch_types=[
        pltpu.VMEM((8, 16), jnp.int32),
        pltpu.VMEM((_BLK,), jnp.float32),
        pltpu.VMEM((_BLK,), jnp.float32),
        pltpu.VMEM((_BLK,), jnp.float32),
        pltpu.VMEM((_BLK,), jnp.float32),
        pltpu.SemaphoreType.DMA,
        pltpu.SemaphoreType.DMA,
        pltpu.SemaphoreType.DMA,
        pltpu.SemaphoreType.DMA,
        pltpu.SemaphoreType.DMA,
        pltpu.SemaphoreType.DMA,
        pltpu.SemaphoreType.DMA,
        pltpu.SemaphoreType.DMA,
    ],
    compiler_params=pltpu.CompilerParams(
        use_tc_tiling_on_sc=True, needs_layout_passes=False
    ),
)


def kernel(x):
    x1 = x.T.reshape(_B * _COLS)
    stab = jnp.zeros((8, 16), jnp.int32).at[:_NSL].set(
        jnp.asarray(_S_STATIC[: _NSL * 16].reshape(_NSL, 16))
    )
    out1 = _blend(x1, stab)
    return out1.reshape(_COLS, _B).T
